# baseline (speedup 1.0000x reference)
import functools

import jax
import jax.numpy as jnp
from jax import lax
from jax.experimental import pallas as pl
from jax.experimental.pallas import tpu as pltpu
from jax.experimental.pallas import tpu_sc as plsc

_NC = 2
_NS = 16
_NW = _NC * _NS
_G = 16


def _embed_body(per_w, table_hbm, idx_hbm, out_hbm, idx_v, rows_v, gsem, osem):
    wid = lax.axis_index("s") * _NC + lax.axis_index("c")
    base = wid * per_w
    n_step = per_w // _G
    pltpu.sync_copy(idx_hbm.at[pl.ds(base, per_w)], idx_v)

    def out_drain(b):
        pltpu.make_async_copy(
            rows_v.at[b], out_hbm.at[pl.ds(base, _G)], osem
        ).wait()

    def body(i, carry):
        for b in range(2):
            s = i * 2 + b

            @pl.when(s >= 2)
            def _():
                out_drain(b)

            iv = idx_v[pl.ds(s * _G, _G)]
            descs = [
                pltpu.make_async_copy(
                    table_hbm.at[pl.ds(iv[k], 1)],
                    rows_v.at[b, pl.ds(k, 1)],
                    gsem,
                )
                for k in range(_G)
            ]
            for dsc in descs:
                dsc.start()
            for dsc in descs:
                dsc.wait()

            pltpu.make_async_copy(
                rows_v.at[b], out_hbm.at[pl.ds(base + s * _G, _G)], osem
            ).start()
        return carry

    lax.fori_loop(0, n_step // 2, body, 0)
    for b in range(2):
        out_drain(b)


@functools.partial(jax.jit, static_argnums=(2,))
def _embed(idx, W, total):
    d = W.shape[1]
    per_w = total // _NW
    mesh = plsc.VectorSubcoreMesh(core_axis_name="c", subcore_axis_name="s")
    k = pl.kernel(
        functools.partial(_embed_body, per_w),
        out_type=jax.ShapeDtypeStruct((total, d), jnp.float32),
        mesh=mesh,
        scratch_types=[
            pltpu.VMEM((per_w,), jnp.int32),
            pltpu.VMEM((2, _G, d), jnp.float32),
            pltpu.SemaphoreType.DMA,
            pltpu.SemaphoreType.DMA,
        ],
    )
    return k(W, idx)


def kernel(x, W):
    batch, seq = x.shape
    idx = x.reshape(-1).astype(jnp.int32)
    out = _embed(idx, W, batch * seq)
    return out.reshape(batch, seq, W.shape[1])


# speedup vs baseline: 1.6857x; 1.6857x over previous
import functools

import jax
import jax.numpy as jnp
from jax import lax
from jax.experimental import pallas as pl
from jax.experimental.pallas import tpu as pltpu
from jax.experimental.pallas import tpu_sc as plsc

_NC = 2
_NS = 16
_NW = _NC * _NS

_R = 4
_SPLITS = ((0, 96), (96, 104))


def _embed_body(rows_per_w, seq, table_hbm, idx_hbm, out_hbm, idx_v, rows_v,
                gsem, osem):
    wid = lax.axis_index("s") * _NC + lax.axis_index("c")
    row0 = wid * rows_per_w
    n_step = rows_per_w // _R
    pltpu.sync_copy(idx_hbm.at[pl.ds(row0 * seq, rows_per_w * seq)], idx_v)

    def out_drain(b):
        pltpu.make_async_copy(
            rows_v.at[b], out_hbm.at[pl.ds(row0, _R)], osem
        ).wait()

    def body(i, carry):
        for b in range(2):
            s = i * 2 + b

            @pl.when(s >= 2)
            def _():
                out_drain(b)

            descs = []
            for r in range(_R):
                for off, ln in _SPLITS:
                    descs.append(pltpu.make_async_copy(
                        table_hbm.at[
                            idx_v.at[pl.ds((s * _R + r) * seq + off, ln)]],
                        rows_v.at[b, r, pl.ds(off, ln)],
                        gsem,
                    ))
            for dsc in descs:
                dsc.start()
            for dsc in descs:
                dsc.wait()

            pltpu.make_async_copy(
                rows_v.at[b], out_hbm.at[pl.ds(row0 + s * _R, _R)], osem
            ).start()
        return carry

    lax.fori_loop(0, n_step // 2, body, 0)
    for b in range(2):
        out_drain(b)


@functools.partial(jax.jit, static_argnums=(2, 3))
def _embed(idx, W, batch, seq):
    d = W.shape[1]
    rows_per_w = batch // _NW
    mesh = plsc.VectorSubcoreMesh(core_axis_name="c", subcore_axis_name="s")
    k = pl.kernel(
        functools.partial(_embed_body, rows_per_w, seq),
        out_type=jax.ShapeDtypeStruct((batch, seq, d), jnp.float32),
        mesh=mesh,
        scratch_types=[
            pltpu.VMEM((rows_per_w * seq,), jnp.int32),
            pltpu.VMEM((2, _R, seq, d), jnp.float32),
            pltpu.SemaphoreType.DMA,
            pltpu.SemaphoreType.DMA,
        ],
        compiler_params=pltpu.CompilerParams(use_tc_tiling_on_sc=False),
    )
    return k(W, idx)


def kernel(x, W):
    batch, seq = x.shape
    idx = x.reshape(-1).astype(jnp.int32)
    return _embed(idx, W, batch, seq)


# speedup vs baseline: 1.6880x; 1.0014x over previous
import functools

import jax
import jax.numpy as jnp
from jax import lax
from jax.experimental import pallas as pl
from jax.experimental.pallas import tpu as pltpu
from jax.experimental.pallas import tpu_sc as plsc

_NC = 2
_NS = 16
_NW = _NC * _NS

_R = 4
_SPLITS = ((0, 96), (96, 104))


def _embed_body(rows_per_w, seq, table_hbm, idx_hbm, out_hbm, idx_v, rows_v,
                gsem, osem):
    wid = lax.axis_index("s") * _NC + lax.axis_index("c")
    row0 = wid * rows_per_w
    n_step = rows_per_w // _R
    pltpu.sync_copy(idx_hbm.at[pl.ds(row0 * seq, rows_per_w * seq)], idx_v)

    def out_drain(b):
        pltpu.make_async_copy(
            rows_v.at[b], out_hbm.at[pl.ds(row0, _R)], osem
        ).wait()

    def body(i, carry):
        for b in range(2):
            s = i * 2 + b

            @pl.when(s >= 2)
            def _():
                out_drain(b)

            descs = []
            for r in range(_R):
                for off, ln in _SPLITS:
                    descs.append(pltpu.make_async_copy(
                        table_hbm.at[
                            idx_v.at[pl.ds((s * _R + r) * seq + off, ln)]],
                        rows_v.at[b, r, pl.ds(off, ln)],
                        gsem,
                    ))
            for dsc in descs:
                dsc.start()
            for dsc in descs:
                dsc.wait()

            pltpu.make_async_copy(
                rows_v.at[b], out_hbm.at[pl.ds(row0 + s * _R, _R)], osem
            ).start()
        return carry

    lax.fori_loop(0, n_step // 2, body, 0)
    for b in range(2):
        out_drain(b)


@functools.partial(jax.jit, static_argnums=(2, 3))
def _embed(idx, W, batch, seq):
    d = W.shape[1]
    rows_per_w = batch // _NW
    mesh = plsc.VectorSubcoreMesh(core_axis_name="c", subcore_axis_name="s")
    k = pl.kernel(
        functools.partial(_embed_body, rows_per_w, seq),
        out_type=jax.ShapeDtypeStruct((batch, seq, d), jnp.float32),
        mesh=mesh,
        scratch_types=[
            pltpu.VMEM((rows_per_w * seq,), jnp.int32),
            pltpu.VMEM((2, _R, seq, d), jnp.float32),
            pltpu.SemaphoreType.DMA,
            pltpu.SemaphoreType.DMA,
        ],
        compiler_params=pltpu.CompilerParams(use_tc_tiling_on_sc=False),
    )
    return k(W, idx)


def kernel(x, W):
    batch, seq = x.shape
    idx = x.reshape(-1).astype(jnp.int32)
    half = batch // 2
    o1 = _embed(idx[: half * seq], W, half, seq)
    o2 = _embed(idx[half * seq:], W, half, seq)
    return jnp.concatenate([o1, o2], axis=0)
